# baseline (speedup 1.0000x reference)
import numpy as np
import jax
import jax.numpy as jnp
from jax import lax
from jax.experimental import pallas as pl
from jax.experimental.pallas import tpu as pltpu
from jax.experimental.pallas import tpu_sc as plsc

R, N, TOPK = 128, 32768, 64
L = 16
NV = N // L
NG = NV // L
NW4 = N // 4
BINS = 4096
HV = BINS // L
CAP = 512
CV = CAP // L
NC, NS = 2, 16
NW = NC * NS
ROWS_PER_W = R // NW

T0B = int(np.float32(2.73).view(np.int32))


def _mono_key(v):
    s = lax.bitcast_convert_type(v, jnp.int32)
    return s ^ ((s >> 31) & jnp.int32(0x7FFFFFFF))


def _key_to_val(k):
    s = k ^ ((k >> 31) & jnp.int32(0x7FFFFFFF))
    return lax.bitcast_convert_type(s, jnp.float32)


def _body(x_hbm, res_hbm, maskw_hbm, idx_hbm,
          row_v, res_st, mw_st, hist_v, pcv, fvid, fbase,
          candk, candi, outk, outi):
    cid = lax.axis_index("c")
    sid = lax.axis_index("s")
    wid = sid * NC + cid

    z16f = jnp.zeros((L,), jnp.float32)
    z16i = jnp.zeros((L,), jnp.int32)
    ones16 = jnp.ones((L,), jnp.int32)
    lanes = lax.iota(jnp.int32, L)

    def _z1(i, c):
        res_st[pl.ds(i * L, L)] = z16f
        return c
    lax.fori_loop(0, NV, _z1, 0, unroll=8)

    def _z1b(i, c):
        mw_st[pl.ds(i * L, L)] = z16i
        return c
    lax.fori_loop(0, NW4 // L, _z1b, 0, unroll=8)

    def _row(j, carry0):
        r = wid * ROWS_PER_W + j
        pltpu.sync_copy(x_hbm.at[r], row_v)

        def _zc(i, c):
            candk[pl.ds(i * L, L)] = jnp.full((L,), jnp.int32(-(2 ** 31)))
            candi[pl.ds(i * L, L)] = jnp.full((L,), jnp.int32(2 ** 31 - 1))
            fvid[pl.ds(i * L, L)] = z16i
            fbase[pl.ds(i * L, L)] = jnp.full((L,), jnp.int32(CAP))
            return c
        lax.fori_loop(0, CV, _zc, 0, unroll=4)

        @plsc.parallel_loop(0, NG)
        def _pa(g):
            acc = z16i
            for e in range(L):
                v = row_v[pl.ds((g * L + e) * L, L)]
                s = lax.bitcast_convert_type(v, jnp.int32)
                m = s >= T0B
                pc = plsc.all_reduce_population_count(m)
                acc = jnp.where(lanes == e, pc, acc)
            pcv[pl.ds(g * L, L)] = acc

        def _pb(g, carry):
            base_s, nf_s = carry
            pc = pcv[pl.ds(g * L, L)]
            csum = plsc.cumsum(pc)
            bases = base_s + csum - pc
            m2 = pc > 0
            m2i = m2.astype(jnp.int32)
            c2 = plsc.cumsum(m2i)
            p2 = nf_s + c2 - m2i
            okm = m2 & (p2 < CAP)
            plsc.store_scatter(fvid, [p2], g * L + lanes, mask=okm)
            plsc.store_scatter(fbase, [p2], bases, mask=okm)
            return (base_s + csum[L - 1], nf_s + c2[L - 1])
        base_s, nf_s = lax.fori_loop(0, NG, _pb, (z16i, z16i))
        cnt = jnp.max(base_s)
        nf = jnp.minimum(jnp.max(nf_s), jnp.int32(CAP))
        good = (cnt >= TOPK) & (cnt <= CAP)

        def _fast():
            nch = (nf + L - 1) // L

            def _pc(ch, c):
                vids = fvid[pl.ds(ch * L, L)]
                bss = fbase[pl.ds(ch * L, L)]
                for e in range(L):
                    addr = vids[e] * L + lanes
                    v = plsc.load_gather(row_v, [addr])
                    s = lax.bitcast_convert_type(v, jnp.int32)
                    m = s >= T0B
                    mi = m.astype(jnp.int32)
                    cs = plsc.cumsum(mi)
                    pos = bss[e] + cs - mi
                    okm = m & (pos < CAP)
                    plsc.store_scatter(candk, [pos], s, mask=okm)
                    plsc.store_scatter(candi, [pos], addr, mask=okm)
                return c
            lax.fori_loop(0, nch, _pc, 0)
            return cnt

        def _slow():
            def _zh(i, c):
                hist_v[pl.ds(i * L, L)] = z16i
                return c
            lax.fori_loop(0, HV, _zh, 0, unroll=8)

            def _h(i, c):
                v = row_v[pl.ds(i * L, L)]
                k = _mono_key(v)
                b = (k >> 20) + (BINS // 2)
                plsc.addupdate_scatter(hist_v, [b], ones16)
                return c
            lax.fori_loop(0, NV, _h, 0, unroll=8)

            def _t(t, carry):
                above, bstar, found = carry
                vb = HV - 1 - t
                h = hist_v[pl.ds(vb * L, L)]
                csum = plsc.cumsum(h)
                tot = jnp.max(csum)
                suffix = (tot - csum + h) + above
                m = suffix >= TOPK
                p = jnp.max(plsc.all_reduce_population_count(m))
                hit = (found == 0) & (p > 0)
                bstar = jnp.where(hit, vb * L + p - 1, bstar)
                found = jnp.where(hit, jnp.int32(1), found)
                return (above + tot, bstar, found)
            _, bstar, _ = lax.fori_loop(
                0, HV, _t, (jnp.int32(0), jnp.int32(0), jnp.int32(0)),
                unroll=4)

            def _c(i, c2):
                v = row_v[pl.ds(i * L, L)]
                k = _mono_key(v)
                b = (k >> 20) + (BINS // 2)
                m = b >= bstar
                mi = m.astype(jnp.int32)
                incl = plsc.cumsum(mi)
                pos = c2 + incl - mi
                mm = m & (pos < CAP)
                plsc.store_scatter(candk, [pos], k, mask=mm)
                plsc.store_scatter(candi, [pos], i * L + lanes, mask=mm)
                return c2 + jnp.max(plsc.all_reduce_population_count(m))
            return lax.fori_loop(0, NV, _c, jnp.int32(0), unroll=8)

        cand_n = lax.cond(good, _fast, _slow)
        csz = jnp.minimum(cand_n, jnp.int32(CAP))
        ndv = (csz + L - 1) // L

        def _q(qv, c):
            qk = candk[pl.ds(qv * L, L)]
            qi = candi[pl.ds(qv * L, L)]

            def _d(dv, rank):
                kd = candk[pl.ds(dv * L, L)]
                idd = candi[pl.ds(dv * L, L)]
                for e in range(L):
                    ke = kd[e]
                    ie = idd[e]
                    beat = (ke > qk) | ((ke == qk) & (ie < qi))
                    rank = rank + beat.astype(jnp.int32)
                return rank
            rank = lax.fori_loop(0, ndv, _d, z16i)
            m = rank < TOPK
            plsc.store_scatter(outk, [rank], qk, mask=m)
            plsc.store_scatter(outi, [rank], qi, mask=m)
            return c
        lax.fori_loop(0, ndv, _q, 0)

        pltpu.sync_copy(outi, idx_hbm.at[r])

        def _v(i, c):
            kk = outk[pl.ds(i * L, L)]
            vv = jnp.maximum(_key_to_val(kk), 0.0)
            ii = outi[pl.ds(i * L, L)]
            plsc.store_scatter(res_st, [ii], vv)
            w = ii >> 2
            bval = jnp.int32(1) << ((ii & 3) * 8)
            plsc.addupdate_scatter(mw_st, [w], bval)
            return c
        lax.fori_loop(0, TOPK // L, _v, 0, unroll=True)
        pltpu.sync_copy(res_st, res_hbm.at[r])
        pltpu.sync_copy(mw_st, maskw_hbm.at[r])

        def _rz(i, c):
            ii = outi[pl.ds(i * L, L)]
            plsc.store_scatter(res_st, [ii], z16f)
            w = ii >> 2
            bval = jnp.int32(1) << ((ii & 3) * 8)
            plsc.addupdate_scatter(mw_st, [w], -bval)
            return c
        lax.fori_loop(0, TOPK // L, _rz, 0, unroll=True)
        return carry0

    lax.fori_loop(0, ROWS_PER_W, _row, 0)


@jax.jit
def kernel(x):
    mesh = plsc.VectorSubcoreMesh(core_axis_name="c", subcore_axis_name="s")
    res, maskb, idx = pl.kernel(
        _body,
        out_type=[
            jax.ShapeDtypeStruct((R, N), jnp.float32),
            jax.ShapeDtypeStruct((R, NW4), jnp.int32),
            jax.ShapeDtypeStruct((R, TOPK), jnp.int32),
        ],
        mesh=mesh,
        compiler_params=pltpu.CompilerParams(needs_layout_passes=False),
        scratch_types=[
            pltpu.VMEM((N,), jnp.float32),
            pltpu.VMEM((N,), jnp.float32),
            pltpu.VMEM((NW4,), jnp.int32),
            pltpu.VMEM((BINS,), jnp.int32),
            pltpu.VMEM((NV,), jnp.int32),
            pltpu.VMEM((CAP,), jnp.int32),
            pltpu.VMEM((CAP,), jnp.int32),
            pltpu.VMEM((CAP,), jnp.int32),
            pltpu.VMEM((CAP,), jnp.int32),
            pltpu.VMEM((TOPK,), jnp.int32),
            pltpu.VMEM((TOPK,), jnp.int32),
        ],
    )(x)
    mask = lax.bitcast_convert_type(maskb, jnp.int8).reshape(R, N) != 0
    return (res, mask, idx)


# speedup vs baseline: 1.2296x; 1.2296x over previous
import numpy as np
import jax
import jax.numpy as jnp
from jax import lax
from jax.experimental import pallas as pl
from jax.experimental.pallas import tpu as pltpu
from jax.experimental.pallas import tpu_sc as plsc

R, N, TOPK = 128, 32768, 64
L = 16
NV = N // L
NG = NV // L
NW4 = N // 4
BINS = 4096
HV = BINS // L
CAP = 512
CV = CAP // L
NC, NS = 2, 16
NW = NC * NS
ROWS_PER_W = R // NW

T0B = int(np.float32(2.73).view(np.int32))


def _mono_key(v):
    s = lax.bitcast_convert_type(v, jnp.int32)
    return s ^ ((s >> 31) & jnp.int32(0x7FFFFFFF))


def _key_to_val(k):
    s = k ^ ((k >> 31) & jnp.int32(0x7FFFFFFF))
    return lax.bitcast_convert_type(s, jnp.float32)


def _body(x_hbm, res_hbm, maskw_hbm, idx_hbm,
          row_v, res_st, mw_st, hist_v, pcv, fvid, fbase,
          candk, candi, outk, outi):
    cid = lax.axis_index("c")
    sid = lax.axis_index("s")
    wid = sid * NC + cid

    z16f = jnp.zeros((L,), jnp.float32)
    z16i = jnp.zeros((L,), jnp.int32)
    ones16 = jnp.ones((L,), jnp.int32)
    lanes = lax.iota(jnp.int32, L)

    def _z1(i, c):
        res_st[pl.ds(i * L, L)] = z16f
        return c
    lax.fori_loop(0, NV, _z1, 0, unroll=8)

    def _z1b(i, c):
        mw_st[pl.ds(i * L, L)] = z16i
        return c
    lax.fori_loop(0, N // L, _z1b, 0, unroll=8)

    def _row(j, carry0):
        r = wid * ROWS_PER_W + j
        pltpu.sync_copy(x_hbm.at[r], row_v)

        def _zc(i, c):
            candk[pl.ds(i * L, L)] = jnp.full((L,), jnp.int32(-(2 ** 31)))
            candi[pl.ds(i * L, L)] = jnp.full((L,), jnp.int32(2 ** 31 - 1))
            fvid[pl.ds(i * L, L)] = z16i
            fbase[pl.ds(i * L, L)] = jnp.full((L,), jnp.int32(CAP))
            return c
        lax.fori_loop(0, CV, _zc, 0, unroll=4)

        @plsc.parallel_loop(0, NG)
        def _pa(g):
            acc = z16i
            for e in range(L):
                v = row_v[pl.ds((g * L + e) * L, L)]
                s = lax.bitcast_convert_type(v, jnp.int32)
                m = s >= T0B
                pc = plsc.all_reduce_population_count(m)
                acc = jnp.where(lanes == e, pc, acc)
            pcv[pl.ds(g * L, L)] = acc

        def _pb(g, carry):
            base_s, nf_s = carry
            pc = pcv[pl.ds(g * L, L)]
            csum = plsc.cumsum(pc)
            bases = base_s + csum - pc
            m2 = pc > 0
            m2i = m2.astype(jnp.int32)
            c2 = plsc.cumsum(m2i)
            p2 = nf_s + c2 - m2i
            okm = m2 & (p2 < CAP)
            plsc.store_scatter(fvid, [p2], g * L + lanes, mask=okm)
            plsc.store_scatter(fbase, [p2], bases, mask=okm)
            return (base_s + csum[L - 1], nf_s + c2[L - 1])
        base_s, nf_s = lax.fori_loop(0, NG, _pb, (z16i, z16i))
        cnt = jnp.max(base_s)
        nf = jnp.minimum(jnp.max(nf_s), jnp.int32(CAP))
        good = (cnt >= TOPK) & (cnt <= CAP)

        def _fast():
            nch = (nf + L - 1) // L

            def _pc(ch, c):
                vids = fvid[pl.ds(ch * L, L)]
                bss = fbase[pl.ds(ch * L, L)]
                for e in range(L):
                    addr = vids[e] * L + lanes
                    v = plsc.load_gather(row_v, [addr])
                    s = lax.bitcast_convert_type(v, jnp.int32)
                    m = s >= T0B
                    mi = m.astype(jnp.int32)
                    cs = plsc.cumsum(mi)
                    pos = bss[e] + cs - mi
                    okm = m & (pos < CAP)
                    plsc.store_scatter(candk, [pos], s, mask=okm)
                    plsc.store_scatter(candi, [pos], addr, mask=okm)
                return c
            lax.fori_loop(0, nch, _pc, 0)
            return cnt

        def _slow():
            def _zh(i, c):
                hist_v[pl.ds(i * L, L)] = z16i
                return c
            lax.fori_loop(0, HV, _zh, 0, unroll=8)

            def _h(i, c):
                v = row_v[pl.ds(i * L, L)]
                k = _mono_key(v)
                b = (k >> 20) + (BINS // 2)
                plsc.addupdate_scatter(hist_v, [b], ones16)
                return c
            lax.fori_loop(0, NV, _h, 0, unroll=8)

            def _t(t, carry):
                above, bstar, found = carry
                vb = HV - 1 - t
                h = hist_v[pl.ds(vb * L, L)]
                csum = plsc.cumsum(h)
                tot = jnp.max(csum)
                suffix = (tot - csum + h) + above
                m = suffix >= TOPK
                p = jnp.max(plsc.all_reduce_population_count(m))
                hit = (found == 0) & (p > 0)
                bstar = jnp.where(hit, vb * L + p - 1, bstar)
                found = jnp.where(hit, jnp.int32(1), found)
                return (above + tot, bstar, found)
            _, bstar, _ = lax.fori_loop(
                0, HV, _t, (jnp.int32(0), jnp.int32(0), jnp.int32(0)),
                unroll=4)

            def _c(i, c2):
                v = row_v[pl.ds(i * L, L)]
                k = _mono_key(v)
                b = (k >> 20) + (BINS // 2)
                m = b >= bstar
                mi = m.astype(jnp.int32)
                incl = plsc.cumsum(mi)
                pos = c2 + incl - mi
                mm = m & (pos < CAP)
                plsc.store_scatter(candk, [pos], k, mask=mm)
                plsc.store_scatter(candi, [pos], i * L + lanes, mask=mm)
                return c2 + jnp.max(plsc.all_reduce_population_count(m))
            return lax.fori_loop(0, NV, _c, jnp.int32(0), unroll=8)

        cand_n = lax.cond(good, _fast, _slow)
        csz = jnp.minimum(cand_n, jnp.int32(CAP))
        ndv = (csz + L - 1) // L

        def _q(qv, c):
            qk = candk[pl.ds(qv * L, L)]
            qi = candi[pl.ds(qv * L, L)]

            def _d(dv, rank):
                kd = candk[pl.ds(dv * L, L)]
                idd = candi[pl.ds(dv * L, L)]
                for e in range(L):
                    ke = kd[e]
                    ie = idd[e]
                    beat = (ke > qk) | ((ke == qk) & (ie < qi))
                    rank = rank + beat.astype(jnp.int32)
                return rank
            rank = lax.fori_loop(0, ndv, _d, z16i)
            m = rank < TOPK
            plsc.store_scatter(outk, [rank], qk, mask=m)
            plsc.store_scatter(outi, [rank], qi, mask=m)
            return c
        lax.fori_loop(0, ndv, _q, 0)

        pltpu.sync_copy(outi, idx_hbm.at[r])

        def _v(i, c):
            kk = outk[pl.ds(i * L, L)]
            vv = jnp.maximum(_key_to_val(kk), 0.0)
            ii = outi[pl.ds(i * L, L)]
            plsc.store_scatter(res_st, [ii], vv)
            plsc.store_scatter(mw_st, [ii], ones16)
            return c
        lax.fori_loop(0, TOPK // L, _v, 0, unroll=True)
        pltpu.sync_copy(res_st, res_hbm.at[r])
        pltpu.sync_copy(mw_st, maskw_hbm.at[r])

        def _rz(i, c):
            ii = outi[pl.ds(i * L, L)]
            plsc.store_scatter(res_st, [ii], z16f)
            plsc.store_scatter(mw_st, [ii], z16i)
            return c
        lax.fori_loop(0, TOPK // L, _rz, 0, unroll=True)
        return carry0

    lax.fori_loop(0, ROWS_PER_W, _row, 0)


@jax.jit
def kernel(x):
    mesh = plsc.VectorSubcoreMesh(core_axis_name="c", subcore_axis_name="s")
    res, maskb, idx = pl.kernel(
        _body,
        out_type=[
            jax.ShapeDtypeStruct((R, N), jnp.float32),
            jax.ShapeDtypeStruct((R, N), jnp.int32),
            jax.ShapeDtypeStruct((R, TOPK), jnp.int32),
        ],
        mesh=mesh,
        compiler_params=pltpu.CompilerParams(needs_layout_passes=False),
        scratch_types=[
            pltpu.VMEM((N,), jnp.float32),
            pltpu.VMEM((N,), jnp.float32),
            pltpu.VMEM((N,), jnp.int32),
            pltpu.VMEM((BINS,), jnp.int32),
            pltpu.VMEM((NV,), jnp.int32),
            pltpu.VMEM((CAP,), jnp.int32),
            pltpu.VMEM((CAP,), jnp.int32),
            pltpu.VMEM((CAP,), jnp.int32),
            pltpu.VMEM((CAP,), jnp.int32),
            pltpu.VMEM((TOPK,), jnp.int32),
            pltpu.VMEM((TOPK,), jnp.int32),
        ],
    )(x)
    return (res, maskb.astype(jnp.bool_), idx)


# speedup vs baseline: 1.4341x; 1.1663x over previous
import numpy as np
import jax
import jax.numpy as jnp
from jax import lax
from jax.experimental import pallas as pl
from jax.experimental.pallas import tpu as pltpu
from jax.experimental.pallas import tpu_sc as plsc

R, N, TOPK = 128, 32768, 64
L = 16
NV = N // L
NG = NV // L
NW4 = N // 4
BINS = 4096
HV = BINS // L
CAP = 512
CV = CAP // L
NC, NS = 2, 16
NW = NC * NS
ROWS_PER_W = R // NW

T0B = int(np.float32(2.73).view(np.int32))


def _mono_key(v):
    s = lax.bitcast_convert_type(v, jnp.int32)
    return s ^ ((s >> 31) & jnp.int32(0x7FFFFFFF))


def _key_to_val(k):
    s = k ^ ((k >> 31) & jnp.int32(0x7FFFFFFF))
    return lax.bitcast_convert_type(s, jnp.float32)


def _body(x_hbm, res_hbm, maskw_hbm, idx_hbm,
          row_v, res_st, mw_st, hist_v, pcv, fvid, fbase,
          candk, candi, outk, outi, sem_in, sem_res, sem_mask):
    cid = lax.axis_index("c")
    sid = lax.axis_index("s")
    wid = sid * NC + cid

    z16f = jnp.zeros((L,), jnp.float32)
    z16i = jnp.zeros((L,), jnp.int32)
    ones16 = jnp.ones((L,), jnp.int32)
    lanes = lax.iota(jnp.int32, L)

    def _z1(i, c):
        res_st[pl.ds(i * L, L)] = z16f
        return c
    lax.fori_loop(0, NV, _z1, 0, unroll=8)

    def _z1b(i, c):
        mw_st[pl.ds(i * L, L)] = z16i
        return c
    lax.fori_loop(0, N // L, _z1b, 0, unroll=8)

    pltpu.async_copy(x_hbm.at[wid * ROWS_PER_W], row_v, sem_in)

    def _row(j, carry0):
        r = wid * ROWS_PER_W + j
        pltpu.make_async_copy(x_hbm.at[r], row_v, sem_in).wait()

        def _zc(i, c):
            candk[pl.ds(i * L, L)] = jnp.full((L,), jnp.int32(-(2 ** 31)))
            candi[pl.ds(i * L, L)] = jnp.full((L,), jnp.int32(2 ** 31 - 1))
            fvid[pl.ds(i * L, L)] = z16i
            fbase[pl.ds(i * L, L)] = jnp.full((L,), jnp.int32(CAP))
            return c
        lax.fori_loop(0, CV, _zc, 0, unroll=4)

        @plsc.parallel_loop(0, NG)
        def _pa(g):
            acc = z16i
            for e in range(L):
                v = row_v[pl.ds((g * L + e) * L, L)]
                s = lax.bitcast_convert_type(v, jnp.int32)
                m = s >= T0B
                pc = plsc.all_reduce_population_count(m)
                acc = jnp.where(lanes == e, pc, acc)
            pcv[pl.ds(g * L, L)] = acc

        def _pb(g, carry):
            base_s, nf_s = carry
            pc = pcv[pl.ds(g * L, L)]
            csum = plsc.cumsum(pc)
            bases = base_s + csum - pc
            m2 = pc > 0
            m2i = m2.astype(jnp.int32)
            c2 = plsc.cumsum(m2i)
            p2 = nf_s + c2 - m2i
            okm = m2 & (p2 < CAP)
            plsc.store_scatter(fvid, [p2], g * L + lanes, mask=okm)
            plsc.store_scatter(fbase, [p2], bases, mask=okm)
            return (base_s + csum[L - 1], nf_s + c2[L - 1])
        base_s, nf_s = lax.fori_loop(0, NG, _pb, (z16i, z16i))
        cnt = jnp.max(base_s)
        nf = jnp.minimum(jnp.max(nf_s), jnp.int32(CAP))
        good = (cnt >= TOPK) & (cnt <= CAP)

        def _fast():
            nch = (nf + L - 1) // L

            def _pc(ch, c):
                vids = fvid[pl.ds(ch * L, L)]
                bss = fbase[pl.ds(ch * L, L)]
                for e in range(L):
                    addr = vids[e] * L + lanes
                    v = plsc.load_gather(row_v, [addr])
                    s = lax.bitcast_convert_type(v, jnp.int32)
                    m = s >= T0B
                    mi = m.astype(jnp.int32)
                    cs = plsc.cumsum(mi)
                    pos = bss[e] + cs - mi
                    okm = m & (pos < CAP)
                    plsc.store_scatter(candk, [pos], s, mask=okm)
                    plsc.store_scatter(candi, [pos], addr, mask=okm)
                return c
            lax.fori_loop(0, nch, _pc, 0)
            return cnt

        def _slow():
            def _zh(i, c):
                hist_v[pl.ds(i * L, L)] = z16i
                return c
            lax.fori_loop(0, HV, _zh, 0, unroll=8)

            def _h(i, c):
                v = row_v[pl.ds(i * L, L)]
                k = _mono_key(v)
                b = (k >> 20) + (BINS // 2)
                plsc.addupdate_scatter(hist_v, [b], ones16)
                return c
            lax.fori_loop(0, NV, _h, 0, unroll=8)

            def _t(t, carry):
                above, bstar, found = carry
                vb = HV - 1 - t
                h = hist_v[pl.ds(vb * L, L)]
                csum = plsc.cumsum(h)
                tot = jnp.max(csum)
                suffix = (tot - csum + h) + above
                m = suffix >= TOPK
                p = jnp.max(plsc.all_reduce_population_count(m))
                hit = (found == 0) & (p > 0)
                bstar = jnp.where(hit, vb * L + p - 1, bstar)
                found = jnp.where(hit, jnp.int32(1), found)
                return (above + tot, bstar, found)
            _, bstar, _ = lax.fori_loop(
                0, HV, _t, (jnp.int32(0), jnp.int32(0), jnp.int32(0)),
                unroll=4)

            def _c(i, c2):
                v = row_v[pl.ds(i * L, L)]
                k = _mono_key(v)
                b = (k >> 20) + (BINS // 2)
                m = b >= bstar
                mi = m.astype(jnp.int32)
                incl = plsc.cumsum(mi)
                pos = c2 + incl - mi
                mm = m & (pos < CAP)
                plsc.store_scatter(candk, [pos], k, mask=mm)
                plsc.store_scatter(candi, [pos], i * L + lanes, mask=mm)
                return c2 + jnp.max(plsc.all_reduce_population_count(m))
            return lax.fori_loop(0, NV, _c, jnp.int32(0), unroll=8)

        cand_n = lax.cond(good, _fast, _slow)
        csz = jnp.minimum(cand_n, jnp.int32(CAP))
        ndv = (csz + L - 1) // L

        @pl.when(j < ROWS_PER_W - 1)
        def _pref():
            pltpu.async_copy(x_hbm.at[r + 1], row_v, sem_in)

        @pl.when(j > 0)
        def _drain():
            pltpu.make_async_copy(res_st, res_hbm.at[r], sem_res).wait()
            pltpu.make_async_copy(mw_st, maskw_hbm.at[r], sem_mask).wait()

            def _rz(i, c):
                ii = outi[pl.ds(i * L, L)]
                plsc.store_scatter(res_st, [ii], z16f)
                plsc.store_scatter(mw_st, [ii], z16i)
                return c
            lax.fori_loop(0, TOPK // L, _rz, 0, unroll=True)

        def _q(qv, c):
            qk = candk[pl.ds(qv * L, L)]
            qi = candi[pl.ds(qv * L, L)]

            def _d(dv, rank):
                kd = candk[pl.ds(dv * L, L)]
                idd = candi[pl.ds(dv * L, L)]
                for e in range(L):
                    ke = kd[e]
                    ie = idd[e]
                    beat = (ke > qk) | ((ke == qk) & (ie < qi))
                    rank = rank + beat.astype(jnp.int32)
                return rank
            rank = lax.fori_loop(0, ndv, _d, z16i)
            m = rank < TOPK
            plsc.store_scatter(outk, [rank], qk, mask=m)
            plsc.store_scatter(outi, [rank], qi, mask=m)
            return c
        lax.fori_loop(0, ndv, _q, 0)

        pltpu.sync_copy(outi, idx_hbm.at[r])

        def _v(i, c):
            kk = outk[pl.ds(i * L, L)]
            vv = jnp.maximum(_key_to_val(kk), 0.0)
            ii = outi[pl.ds(i * L, L)]
            plsc.store_scatter(res_st, [ii], vv)
            plsc.store_scatter(mw_st, [ii], ones16)
            return c
        lax.fori_loop(0, TOPK // L, _v, 0, unroll=True)
        pltpu.async_copy(res_st, res_hbm.at[r], sem_res)
        pltpu.async_copy(mw_st, maskw_hbm.at[r], sem_mask)
        pltpu.sync_copy(outi, idx_hbm.at[r])
        return carry0

    lax.fori_loop(0, ROWS_PER_W, _row, 0)
    last_r = wid * ROWS_PER_W + ROWS_PER_W - 1
    pltpu.make_async_copy(res_st, res_hbm.at[last_r], sem_res).wait()
    pltpu.make_async_copy(mw_st, maskw_hbm.at[last_r], sem_mask).wait()


@jax.jit
def kernel(x):
    mesh = plsc.VectorSubcoreMesh(core_axis_name="c", subcore_axis_name="s")
    res, maskb, idx = pl.kernel(
        _body,
        out_type=[
            jax.ShapeDtypeStruct((R, N), jnp.float32),
            jax.ShapeDtypeStruct((R, N), jnp.int32),
            jax.ShapeDtypeStruct((R, TOPK), jnp.int32),
        ],
        mesh=mesh,
        compiler_params=pltpu.CompilerParams(needs_layout_passes=False),
        scratch_types=[
            pltpu.VMEM((N,), jnp.float32),
            pltpu.VMEM((N,), jnp.float32),
            pltpu.VMEM((N,), jnp.int32),
            pltpu.VMEM((BINS,), jnp.int32),
            pltpu.VMEM((NV,), jnp.int32),
            pltpu.VMEM((CAP,), jnp.int32),
            pltpu.VMEM((CAP,), jnp.int32),
            pltpu.VMEM((CAP,), jnp.int32),
            pltpu.VMEM((CAP,), jnp.int32),
            pltpu.VMEM((TOPK,), jnp.int32),
            pltpu.VMEM((TOPK,), jnp.int32),
            pltpu.SemaphoreType.DMA,
            pltpu.SemaphoreType.DMA,
            pltpu.SemaphoreType.DMA,
        ],
    )(x)
    return (res, maskb.astype(jnp.bool_), idx)
